# baseline (speedup 1.0000x reference)
import functools

import jax
import jax.numpy as jnp
from jax import lax
from jax.experimental import pallas as pl
from jax.experimental.pallas import tpu as pltpu
from jax.experimental.pallas import tpu_sc as plsc

_VOCAB = 1000
_VPAD = 1024
_NG = _VPAD // 128
_TAIL = _VOCAB - 128 * (_NG - 1)
_B = 1024
_T = 50

_info = plsc.get_sparse_core_info()
_NC = _info.num_cores
_NS = _info.num_subcores
_NW = _NC * _NS
_NCHUNK = 4
_CB = _B // _NCHUNK
_BPW = _CB // _NW

_mesh = plsc.VectorSubcoreMesh(core_axis_name="c", subcore_axis_name="s")


@functools.partial(
    pl.kernel,
    mesh=_mesh,
    compiler_params=pltpu.CompilerParams(needs_layout_passes=False),
    out_type=jax.ShapeDtypeStruct((_CB, _T, _VOCAB), jnp.float32),
    scratch_types=[
        pltpu.VMEM((_BPW, _T), jnp.int32),
        pltpu.VMEM((_T, _VOCAB), jnp.float32),
        pltpu.VMEM((_T, _VOCAB), jnp.float32),
        pltpu.VMEM((_T, 128), jnp.float32),
        pltpu.SemaphoreType.DMA,
        pltpu.SemaphoreType.DMA,
        pltpu.SemaphoreType.DMA,
    ],
)
def _gather_kernel(idx_hbm, tabg_hbm, out_hbm, idx_v, bufa, bufb, tail_v,
                   gsem, sema, semb):
    wid = lax.axis_index("s") * _NC + lax.axis_index("c")
    pltpu.sync_copy(idx_hbm.at[wid], idx_v)

    def start_gathers(bb, buf):
        ids = idx_v.at[bb]
        for s in range(_NG - 1):
            pltpu.async_copy(tabg_hbm.at[s].at[ids],
                             buf.at[:, pl.ds(128 * s, 128)], gsem)
        pltpu.async_copy(tabg_hbm.at[_NG - 1].at[ids], tail_v, gsem)

    def wait_gathers(bb, buf):
        ids = idx_v.at[bb]
        for s in range(_NG - 1):
            pltpu.make_async_copy(tabg_hbm.at[s].at[ids],
                                  buf.at[:, pl.ds(128 * s, 128)], gsem).wait()
        pltpu.make_async_copy(tabg_hbm.at[_NG - 1].at[ids], tail_v,
                              gsem).wait()

    def copy_tail(buf):
        base = 128 * (_NG - 1)
        nfull = _TAIL // 16
        rem = _TAIL - 16 * nfull
        lane = lax.iota(jnp.int32, 16)

        def row(r, carry):
            for k in range(nfull):
                buf[r, pl.ds(base + 16 * k, 16)] = tail_v[r, pl.ds(16 * k, 16)]
            x = tail_v[r, pl.ds(16 * nfull, 16)]
            rows = jnp.full((16,), r, jnp.int32)
            cols = lane + (base + 16 * nfull)
            plsc.store_scatter(buf, [rows, cols], x, mask=lane < rem)
            return carry

        lax.fori_loop(0, _T, row, 0)

    def start_scatter(bb, buf, sem):
        pltpu.async_copy(buf, out_hbm.at[wid * _BPW + bb], sem)

    def wait_scatter(bb, buf, sem):
        pltpu.make_async_copy(buf, out_hbm.at[wid * _BPW + bb], sem).wait()

    def process(bb, buf, sem, nxt_buf, nxt_sem, wait_prev, issue_next):
        wait_gathers(bb, buf)
        copy_tail(buf)
        start_scatter(bb, buf, sem)
        if issue_next:
            if wait_prev:
                wait_scatter(bb - 1, nxt_buf, nxt_sem)
            start_gathers(bb + 1, nxt_buf)

    start_gathers(0, bufa)
    process(0, bufa, sema, bufb, semb, False, True)
    process(1, bufb, semb, bufa, sema, True, True)

    def pair(i, carry):
        bb0 = 2 * i
        process(bb0, bufa, sema, bufb, semb, True, True)
        process(bb0 + 1, bufb, semb, bufa, sema, True, True)
        return carry

    lax.fori_loop(1, _BPW // 2 - 1, pair, 0, unroll=False)

    process(_BPW - 2, bufa, sema, bufb, semb, True, True)
    process(_BPW - 1, bufb, semb, bufa, sema, False, False)
    wait_scatter(_BPW - 2, bufa, sema)
    wait_scatter(_BPW - 1, bufb, semb)


_alloc_buf = pl.pallas_call(
    lambda o_ref: None,
    grid=(1,),
    out_specs=pl.BlockSpec((1, 8, 128), lambda i: (0, 0, 0)),
    out_shape=jax.ShapeDtypeStruct((_T, _VOCAB, _B), jnp.float32),
)


_RLB = 128


def _make_relayout(k):
    def body(buf_ref, chunk_ref, out_ref, ca, cb, ta, tb,
             csa, csb, osa, osb):
        del buf_ref
        j = pl.program_id(0)
        lane0 = k * _CB + j * _RLB
        cbufs, csems = (ca, cb), (csa, csb)
        tbufs, osems = (ta, tb), (osa, osb)

        def cp_in(t):
            return pltpu.make_async_copy(
                chunk_ref.at[:, t, :], cbufs[t % 2], csems[t % 2])

        def cp_out(t):
            return pltpu.make_async_copy(
                tbufs[t % 2], out_ref.at[t].at[:, pl.ds(lane0, _RLB)],
                osems[t % 2])

        cp_in(0).start()
        for t in range(_T):
            if t + 1 < _T:
                cp_in(t + 1).start()
            cp_in(t).wait()
            if t >= 2:
                cp_out(t - 2).wait()
            tbufs[t % 2][...] = cbufs[t % 2][...].T
            cp_out(t).start()
        cp_out(_T - 2).wait()
        cp_out(_T - 1).wait()

    return pl.pallas_call(
        body,
        grid=(_CB // _RLB,),
        in_specs=[
            pl.BlockSpec(memory_space=pl.ANY),
            pl.BlockSpec((_RLB, _T, _VOCAB), lambda j: (j, 0, 0)),
        ],
        out_specs=pl.BlockSpec(memory_space=pl.ANY),
        out_shape=jax.ShapeDtypeStruct((_T, _VOCAB, _B), jnp.float32),
        scratch_shapes=[
            pltpu.VMEM((_RLB, _VOCAB), jnp.float32),
            pltpu.VMEM((_RLB, _VOCAB), jnp.float32),
            pltpu.VMEM((_VOCAB, _RLB), jnp.float32),
            pltpu.VMEM((_VOCAB, _RLB), jnp.float32),
            pltpu.SemaphoreType.DMA,
            pltpu.SemaphoreType.DMA,
            pltpu.SemaphoreType.DMA,
            pltpu.SemaphoreType.DMA,
        ],
        input_output_aliases={0: 0},
        compiler_params=pltpu.CompilerParams(
            vmem_limit_bytes=100 * 1024 * 1024),
    )


def kernel(idx, table):
    table_padded = jnp.pad(table, ((0, 0), (0, _VPAD - _VOCAB)))
    tabg = table_padded.reshape(_VOCAB, _NG, 128).swapaxes(0, 1)
    buf = _alloc_buf()
    buf, idx = lax.optimization_barrier((buf, idx))
    idx_w = idx.reshape(_NCHUNK, _NW, _BPW, _T)
    for k in range(_NCHUNK):
        chunk = _gather_kernel(idx_w[k], tabg)
        buf = _make_relayout(k)(buf, chunk)
    return jnp.transpose(buf, (2, 0, 1))


# speedup vs baseline: 1.1844x; 1.1844x over previous
import functools

import jax
import jax.numpy as jnp
from jax import lax
from jax.experimental import pallas as pl
from jax.experimental.pallas import tpu as pltpu
from jax.experimental.pallas import tpu_sc as plsc

_VOCAB = 1000
_VPAD = 1024
_NG = _VPAD // 128
_TAIL = _VOCAB - 128 * (_NG - 1)
_B = 1024
_T = 50

_info = plsc.get_sparse_core_info()
_NC = _info.num_cores
_NS = _info.num_subcores
_NW = _NC * _NS
_NCHUNK = 4
_CB = _B // _NCHUNK
_BPW = _CB // _NW

_mesh = plsc.VectorSubcoreMesh(core_axis_name="c", subcore_axis_name="s")


@functools.partial(
    pl.kernel,
    mesh=_mesh,
    compiler_params=pltpu.CompilerParams(needs_layout_passes=False),
    out_type=jax.ShapeDtypeStruct((_CB, _T, _VOCAB), jnp.float32),
    scratch_types=[
        pltpu.VMEM((_BPW, _T), jnp.int32),
        pltpu.VMEM((_T, _VOCAB), jnp.float32),
        pltpu.VMEM((_T, _VOCAB), jnp.float32),
        pltpu.VMEM((_T, 128), jnp.float32),
        pltpu.SemaphoreType.DMA,
        pltpu.SemaphoreType.DMA,
        pltpu.SemaphoreType.DMA,
    ],
)
def _gather_kernel(idx_hbm, tabg_hbm, out_hbm, idx_v, bufa, bufb, tail_v,
                   gsem, sema, semb):
    wid = lax.axis_index("s") * _NC + lax.axis_index("c")
    pltpu.sync_copy(idx_hbm.at[wid], idx_v)

    def start_gathers(bb, buf):
        ids = idx_v.at[bb]
        for s in range(_NG - 1):
            pltpu.async_copy(tabg_hbm.at[s].at[ids],
                             buf.at[:, pl.ds(128 * s, 128)], gsem)
        pltpu.async_copy(tabg_hbm.at[_NG - 1].at[ids], tail_v, gsem)

    def wait_gathers(bb, buf):
        ids = idx_v.at[bb]
        for s in range(_NG - 1):
            pltpu.make_async_copy(tabg_hbm.at[s].at[ids],
                                  buf.at[:, pl.ds(128 * s, 128)], gsem).wait()
        pltpu.make_async_copy(tabg_hbm.at[_NG - 1].at[ids], tail_v,
                              gsem).wait()

    def copy_tail(buf):
        base = 128 * (_NG - 1)
        nfull = _TAIL // 16
        rem = _TAIL - 16 * nfull
        lane = lax.iota(jnp.int32, 16)

        def row(r, carry):
            for k in range(nfull):
                buf[r, pl.ds(base + 16 * k, 16)] = tail_v[r, pl.ds(16 * k, 16)]
            x = tail_v[r, pl.ds(16 * nfull, 16)]
            rows = jnp.full((16,), r, jnp.int32)
            cols = lane + (base + 16 * nfull)
            plsc.store_scatter(buf, [rows, cols], x, mask=lane < rem)
            return carry

        lax.fori_loop(0, _T, row, 0)

    def start_scatter(bb, buf, sem):
        pltpu.async_copy(buf, out_hbm.at[wid * _BPW + bb], sem)

    def wait_scatter(bb, buf, sem):
        pltpu.make_async_copy(buf, out_hbm.at[wid * _BPW + bb], sem).wait()

    def process(bb, buf, sem, nxt_buf, nxt_sem, wait_prev, issue_next):
        wait_gathers(bb, buf)
        copy_tail(buf)
        start_scatter(bb, buf, sem)
        if issue_next:
            if wait_prev:
                wait_scatter(bb - 1, nxt_buf, nxt_sem)
            start_gathers(bb + 1, nxt_buf)

    start_gathers(0, bufa)
    process(0, bufa, sema, bufb, semb, False, True)
    process(1, bufb, semb, bufa, sema, True, True)

    def pair(i, carry):
        bb0 = 2 * i
        process(bb0, bufa, sema, bufb, semb, True, True)
        process(bb0 + 1, bufb, semb, bufa, sema, True, True)
        return carry

    lax.fori_loop(1, _BPW // 2 - 1, pair, 0, unroll=False)

    process(_BPW - 2, bufa, sema, bufb, semb, True, True)
    process(_BPW - 1, bufb, semb, bufa, sema, False, False)
    wait_scatter(_BPW - 2, bufa, sema)
    wait_scatter(_BPW - 1, bufb, semb)


_alloc_buf = pl.pallas_call(
    lambda o_ref: None,
    grid=(1,),
    out_specs=pl.BlockSpec((1, 8, 128), lambda i: (0, 0, 0)),
    out_shape=jax.ShapeDtypeStruct((_T, _VOCAB, _B), jnp.float32),
)


_RLB = 128


def _make_relayout(k):
    def body(buf_ref, chunk_ref, out_ref, c0, c1, c2, c3, ta, tb,
             cs0, cs1, cs2, cs3, osa, osb):
        del buf_ref
        j = pl.program_id(0)
        lane0 = k * _CB + j * _RLB
        cbufs, csems = (c0, c1, c2, c3), (cs0, cs1, cs2, cs3)
        tbufs, osems = (ta, tb), (osa, osb)

        def cp_in(t):
            return pltpu.make_async_copy(
                chunk_ref.at[pl.ds(j * _RLB, _RLB), t, :],
                cbufs[t % 4], csems[t % 4])

        def cp_out(t):
            return pltpu.make_async_copy(
                tbufs[t % 2], out_ref.at[t].at[:, pl.ds(lane0, _RLB)],
                osems[t % 2])

        for t in range(3):
            cp_in(t).start()
        for t in range(_T):
            if t + 3 < _T:
                cp_in(t + 3).start()
            cp_in(t).wait()
            if t >= 2:
                cp_out(t - 2).wait()
            tbufs[t % 2][...] = cbufs[t % 4][...].T
            cp_out(t).start()
        cp_out(_T - 2).wait()
        cp_out(_T - 1).wait()

    return pl.pallas_call(
        body,
        grid=(_CB // _RLB,),
        in_specs=[
            pl.BlockSpec(memory_space=pl.ANY),
            pl.BlockSpec(memory_space=pl.ANY),
        ],
        out_specs=pl.BlockSpec(memory_space=pl.ANY),
        out_shape=jax.ShapeDtypeStruct((_T, _VOCAB, _B), jnp.float32),
        scratch_shapes=[
            pltpu.VMEM((_RLB, _VOCAB), jnp.float32),
            pltpu.VMEM((_RLB, _VOCAB), jnp.float32),
            pltpu.VMEM((_RLB, _VOCAB), jnp.float32),
            pltpu.VMEM((_RLB, _VOCAB), jnp.float32),
            pltpu.VMEM((_VOCAB, _RLB), jnp.float32),
            pltpu.VMEM((_VOCAB, _RLB), jnp.float32),
            pltpu.SemaphoreType.DMA,
            pltpu.SemaphoreType.DMA,
            pltpu.SemaphoreType.DMA,
            pltpu.SemaphoreType.DMA,
            pltpu.SemaphoreType.DMA,
            pltpu.SemaphoreType.DMA,
        ],
        input_output_aliases={0: 0},
        compiler_params=pltpu.CompilerParams(
            vmem_limit_bytes=100 * 1024 * 1024),
    )


def kernel(idx, table):
    table_padded = jnp.pad(table, ((0, 0), (0, _VPAD - _VOCAB)))
    tabg = table_padded.reshape(_VOCAB, _NG, 128).swapaxes(0, 1)
    buf = _alloc_buf()
    buf, idx = lax.optimization_barrier((buf, idx))
    idx_w = idx.reshape(_NCHUNK, _NW, _BPW, _T)
    for k in range(_NCHUNK):
        chunk = _gather_kernel(idx_w[k], tabg)
        buf = _make_relayout(k)(buf, chunk)
    return jnp.transpose(buf, (2, 0, 1))
